# baseline (speedup 1.0000x reference)
import functools

import jax
import jax.numpy as jnp
from jax import lax
from jax.experimental import pallas as pl
from jax.experimental.pallas import tpu as pltpu
from jax.experimental.pallas import tpu_sc as plsc

USERS = 2500
ITEMS = 7500
N = USERS + ITEMS
E = 320000
D = 128
LAYERS = 3
KEEP_PROB = 0.9

NC = 2
NS = 16
NW = NC * NS
B = 64
EPT = 9984
NCH = EPT // B
WPT = EPT // 2
NEX = E - NW * EPT
DEPTH = 4

WB = 624


def _spmm_body(e_hbm, colp_hbm, rowp_hbm, valp_hbm, cex_hbm, rex_hbm,
               vex_hbm, p_hbm,
               r0, r1, r2, r3, ci0, ci1, ci2, ci3, wi0, wi1, wi2, wi3,
               colp, rowp, valp, cex, rex, vex, acc,
               g0, g1, g2, g3, s0, s1, s2, s3):
    cid = lax.axis_index("c")
    sid = lax.axis_index("s")
    wid = sid * NC + cid
    rows = (r0, r1, r2, r3)
    colI = (ci0, ci1, ci2, ci3)
    rowI = (wi0, wi1, wi2, wi3)
    gsem = (g0, g1, g2, g3)
    ssem = (s0, s1, s2, s3)

    wsl = pl.ds(pl.multiple_of(wid * WPT, 8), WPT)
    pltpu.sync_copy(colp_hbm.at[wsl], colp)
    pltpu.sync_copy(rowp_hbm.at[wsl], rowp)
    pltpu.sync_copy(valp_hbm.at[wsl], valp)

    zero16 = jnp.zeros((16,), jnp.float32)

    def zrow(i, carry):
        for c in range(D // 16):
            r0[i, pl.ds(c * 16, 16)] = zero16
        return carry

    lax.fori_loop(0, B, zrow, 0)

    NZ = N // B
    NZR = NZ // NS

    def zcopy(k, carry):
        off = pl.multiple_of((sid + k * NS) * B, 8)
        pltpu.sync_copy(r0, acc.at[pl.ds(off, B)])
        return carry

    lax.fori_loop(0, NZR, zcopy, 0)

    @pl.when(sid < NZ - NZR * NS)
    def _():
        off = pl.multiple_of((NZR * NS + sid) * B, 8)
        pltpu.sync_copy(r0, acc.at[pl.ds(off, B)])

    @pl.when(sid == 0)
    def _():
        pltpu.sync_copy(r0.at[pl.ds(0, N - NZ * B)],
                        acc.at[pl.ds(NZ * B, N - NZ * B)])

    plsc.subcore_barrier()


    def _exti(x, par):
        return (x & 0xFFFF) if par == 0 else lax.shift_right_logical(x, 16)

    def _extf(x, par):
        vi = (x & 0xFFFF) if par == 0 else lax.shift_right_logical(x, 16)
        return vi.astype(jnp.float32) * (1.0 / 262144.0)

    def convert_idx(woff, par, b):
        def cbody(j, carry):
            sl = pl.ds(pl.multiple_of(woff + j * 16, 16), 16)
            osl = pl.ds(pl.multiple_of(j * 16, 16), 16)
            colI[b][osl] = _exti(colp[sl], par)
            rowI[b][osl] = _exti(rowp[sl], par)
            return carry

        lax.fori_loop(0, 4, cbody, 0)

    def start_gather(b):
        pltpu.async_copy(e_hbm.at[colI[b]], rows[b], gsem[b])

    def wait_gather(b):
        pltpu.make_async_copy(e_hbm.at[colI[b]], rows[b], gsem[b]).wait()

    def start_scatter(b):
        pltpu.async_copy(rows[b], acc.at[rowI[b]], ssem[b], add=True)

    def wait_scatter(b):
        pltpu.make_async_copy(rows[b], acc.at[rowI[b]], ssem[b]).wait()

    def scale(woff, par, b):
        def sbody(j, carry, _rb=rows[b]):
            sl = pl.ds(pl.multiple_of(woff + j * 16, 16), 16)
            vv = _extf(valp[sl], par)
            for k in range(16):
                i = j * 16 + k
                v = vv[k]
                for c in range(D // 16):
                    csl = pl.ds(c * 16, 16)
                    _rb[i, csl] = _rb[i, csl] * v
            return carry

        lax.fori_loop(0, 4, sbody, 0)

    def _woff(t, b):
        return pl.multiple_of(t * 128 + (b // 2) * B, 16)

    for b in range(2):
        convert_idx(_woff(0, b), b % 2, b)
        start_gather(b)

    def step(t, g, b):
        b2 = (b + 2) % DEPTH

        @pl.when(g >= 2)
        def _():
            wait_scatter(b2)

        @pl.when(g + 2 < NCH)
        def _():
            woff2 = pl.multiple_of(t * 128 + ((b + 2) // 2) * B, 16)
            convert_idx(woff2, b % 2, b2)

        plsc.subcore_barrier()
        wait_gather(b)
        scale(_woff(t, b), b % 2, b)
        start_scatter(b)

        @pl.when(g + 2 < NCH)
        def _():
            start_gather(b2)

    def pass_body(t, carry):
        for b in range(DEPTH):
            step(t, t * DEPTH + b, b)
        return carry

    lax.fori_loop(0, NCH // DEPTH, pass_body, 0)
    for g in range(NCH - 2, NCH):
        wait_scatter(g % DEPTH)

    @pl.when(wid < NEX // 128)
    def _():
        exsl = pl.ds(pl.multiple_of(wid * B, 8), B)
        pltpu.sync_copy(cex_hbm.at[exsl], cex)
        pltpu.sync_copy(rex_hbm.at[exsl], rex)
        pltpu.sync_copy(vex_hbm.at[exsl], vex)
        for par in range(2):
            for j in range(4):
                sl = pl.ds(j * 16, 16)
                colI[par][sl] = _exti(cex[sl], par)
                rowI[par][sl] = _exti(rex[sl], par)
            pltpu.async_copy(e_hbm.at[colI[par]], rows[par],
                             gsem[par]).wait()
            for j in range(4):
                vv = _extf(vex[pl.ds(j * 16, 16)], par)
                for k in range(16):
                    i = j * 16 + k
                    v = vv[k]
                    for c in range(D // 16):
                        csl = pl.ds(c * 16, 16)
                        rows[par][i, csl] = rows[par][i, csl] * v
            pltpu.sync_copy(rows[par], acc.at[rowI[par]], add=True)

    plsc.subcore_barrier()

    off = pl.multiple_of(sid * WB, 16)
    pltpu.sync_copy(acc.at[pl.ds(off, WB)], p_hbm.at[cid, pl.ds(off, WB)])

    @pl.when(sid == NS - 1)
    def _():
        tail = NS * WB
        pltpu.sync_copy(acc.at[pl.ds(tail, N - tail)],
                        p_hbm.at[cid, pl.ds(tail, N - tail)])


@functools.cache
def _get_spmm():
    mesh = plsc.VectorSubcoreMesh(
        core_axis_name="c", subcore_axis_name="s",
        num_cores=NC, num_subcores=NS)
    return pl.kernel(
        _spmm_body,
        out_type=jax.ShapeDtypeStruct((NC, N, D), jnp.float32),
        mesh=mesh,
        scratch_types=(
            [pltpu.VMEM((B, D), jnp.float32) for _ in range(DEPTH)]
            + [pltpu.VMEM((B,), jnp.int32) for _ in range(DEPTH)]
            + [pltpu.VMEM((B,), jnp.int32) for _ in range(DEPTH)]
            + [
                pltpu.VMEM((WPT,), jnp.int32),
                pltpu.VMEM((WPT,), jnp.int32),
                pltpu.VMEM((WPT,), jnp.int32),
                pltpu.VMEM((B,), jnp.int32),
                pltpu.VMEM((B,), jnp.int32),
                pltpu.VMEM((B,), jnp.int32),
                pltpu.VMEM_SHARED((N, D), jnp.float32),
            ]
            + [pltpu.SemaphoreType.DMA for _ in range(2 * DEPTH)]
        ),
    )


_PR = E // (2 * D)


def _pack_body(col_ref, row_ref, rnd_ref, val_ref, cp_ref, rp_ref, vp_ref):
    keep_lo = (rnd_ref[0] + KEEP_PROB).astype(jnp.int32).astype(bool)
    keep_hi = (rnd_ref[1] + KEEP_PROB).astype(jnp.int32).astype(bool)
    val_lo = jnp.where(keep_lo, val_ref[0] * (1.0 / KEEP_PROB), 0.0)
    val_hi = jnp.where(keep_hi, val_ref[1] * (1.0 / KEEP_PROB), 0.0)
    cp_ref[...] = col_ref[0] | lax.shift_left(col_ref[1], 16)
    rp_ref[...] = row_ref[0] | lax.shift_left(row_ref[1], 16)
    flo = (val_lo * 262144.0 + 0.5).astype(jnp.int32)
    fhi = (val_hi * 262144.0 + 0.5).astype(jnp.int32)
    vp_ref[...] = flo | lax.shift_left(fhi, 16)


_pack = pl.pallas_call(
    _pack_body,
    out_shape=(
        jax.ShapeDtypeStruct((_PR, D), jnp.int32),
        jax.ShapeDtypeStruct((_PR, D), jnp.int32),
        jax.ShapeDtypeStruct((_PR, D), jnp.int32),
    ),
    grid=(1,),
    in_specs=[pl.BlockSpec((2, _PR, D), lambda i: (0, 0, 0))] * 4,
    out_specs=[pl.BlockSpec((_PR, D), lambda i: (0, 0))] * 3,
)


_BLK = 400


def _combine_body(p_ref, o_ref):
    o_ref[...] = p_ref[0] + p_ref[1]


_combine = pl.pallas_call(
    _combine_body,
    out_shape=jax.ShapeDtypeStruct((N, D), jnp.float32),
    grid=(N // _BLK,),
    in_specs=[pl.BlockSpec((NC, _BLK, D), lambda i: (0, i, 0))],
    out_specs=pl.BlockSpec((_BLK, D), lambda i: (i, 0)),
)


def _final_body(e0_ref, e1_ref, e2_ref, p_ref, o_ref):
    o_ref[...] = (e0_ref[...] + e1_ref[...] + e2_ref[...]
                  + p_ref[0] + p_ref[1]) * (1.0 / (LAYERS + 1))


_final = pl.pallas_call(
    _final_body,
    out_shape=jax.ShapeDtypeStruct((N, D), jnp.float32),
    grid=(N // _BLK,),
    in_specs=[
        pl.BlockSpec((_BLK, D), lambda i: (i, 0)),
        pl.BlockSpec((_BLK, D), lambda i: (i, 0)),
        pl.BlockSpec((_BLK, D), lambda i: (i, 0)),
        pl.BlockSpec((NC, _BLK, D), lambda i: (0, i, 0)),
    ],
    out_specs=pl.BlockSpec((_BLK, D), lambda i: (i, 0)),
)


def kernel(embedUser, embedItem, graph_row, graph_col, graph_vals):
    rnd = jax.random.uniform(jax.random.key(123), graph_vals.shape)

    col2 = graph_col.astype(jnp.int32).reshape(2, _PR, D)
    row2 = graph_row.astype(jnp.int32).reshape(2, _PR, D)
    rnd2 = rnd.reshape(2, _PR, D)
    val2 = graph_vals.reshape(2, _PR, D)
    cp, rp, vp = _pack(col2, row2, rnd2, val2)
    cp = cp.reshape(E // 2)
    rp = rp.reshape(E // 2)
    vp = vp.reshape(E // 2)
    nmain = NW * WPT
    cpm = cp[:nmain]
    rpm = rp[:nmain]
    vpm = vp[:nmain]
    cex = cp[nmain:]
    rex = rp[nmain:]
    vex = vp[nmain:]

    e0 = jnp.concatenate([embedUser, embedItem], axis=0)

    spmm = _get_spmm()
    e = e0
    embeds = [e0]
    p = None
    for l in range(LAYERS):
        p = spmm(e, cpm, rpm, vpm, cex, rex, vex)
        if l < LAYERS - 1:
            e = _combine(p)
            embeds.append(e)
    out = _final(embeds[0], embeds[1], embeds[2], p)
    return out[:USERS], out[USERS:]


# speedup vs baseline: 1.1400x; 1.1400x over previous
import functools

import jax
import jax.numpy as jnp
from jax import lax
from jax.experimental import pallas as pl
from jax.experimental.pallas import tpu as pltpu
from jax.experimental.pallas import tpu_sc as plsc

USERS = 2500
ITEMS = 7500
N = USERS + ITEMS
E = 320000
D = 128
LAYERS = 3
KEEP_PROB = 0.9

NC = 2
NS = 16
NW = NC * NS
EPT = E // NW
B = 96
NCH = EPT // B
TAIL = EPT - NCH * B
DEPTH = 4

WB = 624


def _spmm_body(e_hbm, col_hbm, row_hbm, val_hbm, p_hbm,
               r0, r1, r2, r3, c0, c1, c2, c3, w0, w1, w2, w3,
               v0, v1, v2, v3, trow, acc,
               g0, g1, g2, g3, s0, s1, s2, s3,
               i0, i1, i2, i3, u0, u1, u2, u3):
    cid = lax.axis_index("c")
    sid = lax.axis_index("s")
    wid = sid * NC + cid
    rows = (r0, r1, r2, r3)
    colc = (c0, c1, c2, c3)
    rowc = (w0, w1, w2, w3)
    valc = (v0, v1, v2, v3)
    gsem = (g0, g1, g2, g3)
    ssem = (s0, s1, s2, s3)
    isem = (i0, i1, i2, i3)
    vsem = (u0, u1, u2, u3)

    ebase = wid * EPT

    def _csl(g, n=B):
        return pl.ds(pl.multiple_of(ebase + g * B, 8), n)

    zero16 = jnp.zeros((16,), jnp.float32)

    def zrow(i, carry):
        for c in range(D // 16):
            r0[i, pl.ds(c * 16, 16)] = zero16
        return carry

    lax.fori_loop(0, B, zrow, 0)

    def zcopy(k, carry):
        off = pl.multiple_of((sid + k * NS) * B, 8)
        pltpu.sync_copy(r0, acc.at[pl.ds(off, B)])
        return carry

    ZCH = N // B
    lax.fori_loop(0, ZCH // NS, zcopy, 0)

    @pl.when(sid < ZCH - (ZCH // NS) * NS)
    def _():
        off = pl.multiple_of(((ZCH // NS) * NS + sid) * B, 8)
        pltpu.sync_copy(r0, acc.at[pl.ds(off, B)])

    @pl.when(sid == 0)
    def _():
        pltpu.sync_copy(r0.at[pl.ds(0, N - ZCH * B)],
                        acc.at[pl.ds(ZCH * B, N - ZCH * B)])

    plsc.subcore_barrier()

    def start_col(g, b):
        pltpu.async_copy(col_hbm.at[_csl(g)], colc[b], isem[b])

    def wait_col(g, b):
        pltpu.make_async_copy(col_hbm.at[_csl(g)], colc[b], isem[b]).wait()

    def start_rowval(g, b):
        pltpu.async_copy(row_hbm.at[_csl(g)], rowc[b], vsem[b])
        pltpu.async_copy(val_hbm.at[_csl(g)], valc[b], vsem[b])

    def wait_rowval(g, b):
        pltpu.make_async_copy(row_hbm.at[_csl(g)], rowc[b], vsem[b]).wait()
        pltpu.make_async_copy(val_hbm.at[_csl(g)], valc[b], vsem[b]).wait()

    def start_gather(g, b):
        pltpu.async_copy(e_hbm.at[colc[b]], rows[b], gsem[b])

    def wait_gather(g, b):
        pltpu.make_async_copy(e_hbm.at[colc[b]], rows[b], gsem[b]).wait()

    def start_scatter(b):
        pltpu.async_copy(rows[b], acc.at[rowc[b]], ssem[b], add=True)

    def wait_scatter(b):
        pltpu.make_async_copy(rows[b], acc.at[rowc[b]], ssem[b]).wait()

    def scale(b, nedge):
        def scale_grp(j, c2, _rb=rows[b], _vc=valc[b]):
            vv = _vc[pl.ds(pl.multiple_of(j * 16, 16), 16)]
            for k in range(16):
                i = j * 16 + k
                v = vv[k]
                for c in range(D // 16):
                    sl = pl.ds(c * 16, 16)
                    _rb[i, sl] = _rb[i, sl] * v
            return c2

        lax.fori_loop(0, nedge // 16, scale_grp, 0)

    for g in range(3):
        start_col(g, g)
    for g in range(2):
        start_rowval(g, g)
    for g in range(2):
        wait_col(g, g)
        start_gather(g, g)

    def step(g, b):
        wait_gather(g, b)
        wait_rowval(g, b)
        scale(b, B)
        start_scatter(b)
        b2 = (b + 2) % DEPTH
        b3 = (b + 3) % DEPTH

        @pl.when(g >= 2)
        def _():
            wait_scatter(b2)

        @pl.when(g + 2 < NCH)
        def _():
            start_rowval(g + 2, b2)
            wait_col(g + 2, b2)
            start_gather(g + 2, b2)

        @pl.when(g + 3 < NCH)
        def _():
            start_col(g + 3, b3)

    def pass_body(t, carry):
        for b in range(DEPTH):
            step(t * DEPTH + b, b)
        return carry

    lax.fori_loop(0, NCH // DEPTH, pass_body, 0)
    for g in range(NCH - 2, NCH):
        wait_scatter(g % DEPTH)

    tsl = _csl(NCH, TAIL)
    pltpu.sync_copy(col_hbm.at[tsl], c0.at[pl.ds(0, TAIL)])
    pltpu.sync_copy(row_hbm.at[tsl], trow)
    pltpu.sync_copy(val_hbm.at[tsl], v0.at[pl.ds(0, TAIL)])
    pltpu.async_copy(e_hbm.at[c0.at[pl.ds(0, TAIL)]],
                     r0.at[pl.ds(0, TAIL)], g0).wait()
    vv = v0[pl.ds(0, TAIL)]
    for k in range(TAIL):
        v = vv[k]
        for c in range(D // 16):
            sl = pl.ds(c * 16, 16)
            r0[k, sl] = r0[k, sl] * v
    pltpu.sync_copy(r0.at[pl.ds(0, TAIL)], acc.at[trow], add=True)

    plsc.subcore_barrier()

    off = pl.multiple_of(sid * WB, 16)
    pltpu.sync_copy(acc.at[pl.ds(off, WB)], p_hbm.at[cid, pl.ds(off, WB)])

    @pl.when(sid == NS - 1)
    def _():
        tail = NS * WB
        pltpu.sync_copy(acc.at[pl.ds(tail, N - tail)],
                        p_hbm.at[cid, pl.ds(tail, N - tail)])


@functools.cache
def _get_spmm():
    mesh = plsc.VectorSubcoreMesh(
        core_axis_name="c", subcore_axis_name="s",
        num_cores=NC, num_subcores=NS)
    return pl.kernel(
        _spmm_body,
        out_type=jax.ShapeDtypeStruct((NC, N, D), jnp.float32),
        mesh=mesh,
        scratch_types=(
            [pltpu.VMEM((B, D), jnp.float32) for _ in range(DEPTH)]
            + [pltpu.VMEM((B,), jnp.int32) for _ in range(DEPTH)]
            + [pltpu.VMEM((B,), jnp.int32) for _ in range(DEPTH)]
            + [pltpu.VMEM((B,), jnp.float32) for _ in range(DEPTH)]
            + [pltpu.VMEM((TAIL,), jnp.int32)]
            + [pltpu.VMEM_SHARED((N, D), jnp.float32)]
            + [pltpu.SemaphoreType.DMA for _ in range(4 * DEPTH)]
        ),
    )


_BLK = 400


def _combine_body(p_ref, o_ref):
    o_ref[...] = p_ref[0] + p_ref[1]


_combine = pl.pallas_call(
    _combine_body,
    out_shape=jax.ShapeDtypeStruct((N, D), jnp.float32),
    grid=(N // _BLK,),
    in_specs=[pl.BlockSpec((NC, _BLK, D), lambda i: (0, i, 0))],
    out_specs=pl.BlockSpec((_BLK, D), lambda i: (i, 0)),
)


def _final_body(e0_ref, e1_ref, e2_ref, p_ref, o_ref):
    o_ref[...] = (e0_ref[...] + e1_ref[...] + e2_ref[...]
                  + p_ref[0] + p_ref[1]) * (1.0 / (LAYERS + 1))


_final = pl.pallas_call(
    _final_body,
    out_shape=jax.ShapeDtypeStruct((N, D), jnp.float32),
    grid=(N // _BLK,),
    in_specs=[
        pl.BlockSpec((_BLK, D), lambda i: (i, 0)),
        pl.BlockSpec((_BLK, D), lambda i: (i, 0)),
        pl.BlockSpec((_BLK, D), lambda i: (i, 0)),
        pl.BlockSpec((NC, _BLK, D), lambda i: (0, i, 0)),
    ],
    out_specs=pl.BlockSpec((_BLK, D), lambda i: (i, 0)),
)


def kernel(embedUser, embedItem, graph_row, graph_col, graph_vals):
    rnd = jax.random.uniform(jax.random.key(123), graph_vals.shape)
    keep = (rnd + KEEP_PROB).astype(jnp.int32).astype(bool)
    vals = jnp.where(keep, graph_vals / KEEP_PROB, 0.0)

    e0 = jnp.concatenate([embedUser, embedItem], axis=0)
    col = graph_col.astype(jnp.int32)
    row = graph_row.astype(jnp.int32)

    spmm = _get_spmm()
    e = e0
    embeds = [e0]
    p = None
    for l in range(LAYERS):
        p = spmm(e, col, row, vals)
        if l < LAYERS - 1:
            e = _combine(p)
            embeds.append(e)
    out = _final(embeds[0], embeds[1], embeds[2], p)
    return out[:USERS], out[USERS:]


# speedup vs baseline: 1.1467x; 1.0058x over previous
import functools

import jax
import jax.numpy as jnp
from jax import lax
from jax.experimental import pallas as pl
from jax.experimental.pallas import tpu as pltpu
from jax.experimental.pallas import tpu_sc as plsc

USERS = 2500
ITEMS = 7500
N = USERS + ITEMS
E = 320000
D = 128
LAYERS = 3
KEEP_PROB = 0.9

NC = 2
NS = 16
NW = NC * NS
EPT = E // NW
B = 128
NCH = EPT // B
TAIL = EPT - NCH * B
DEPTH = 3

WB = 624


def _spmm_body(e_hbm, col_hbm, row_hbm, val_hbm, p_hbm,
               r0, r1, r2, c0, c1, c2, w0, w1, w2,
               v0, v1, v2, trow, acc,
               g0, g1, g2, s0, s1, s2,
               i0, i1, i2, u0, u1, u2):
    cid = lax.axis_index("c")
    sid = lax.axis_index("s")
    wid = sid * NC + cid
    rows = (r0, r1, r2)
    colc = (c0, c1, c2)
    rowc = (w0, w1, w2)
    valc = (v0, v1, v2)
    gsem = (g0, g1, g2)
    ssem = (s0, s1, s2)
    isem = (i0, i1, i2)
    vsem = (u0, u1, u2)

    ebase = wid * EPT

    def _csl(g, n=B):
        return pl.ds(pl.multiple_of(ebase + g * B, 8), n)

    zero16 = jnp.zeros((16,), jnp.float32)

    def zrow(i, carry):
        for c in range(D // 16):
            r0[i, pl.ds(c * 16, 16)] = zero16
        return carry

    lax.fori_loop(0, B, zrow, 0)

    def zcopy(k, carry):
        off = pl.multiple_of((sid + k * NS) * B, 8)
        pltpu.sync_copy(r0, acc.at[pl.ds(off, B)])
        return carry

    ZCH = N // B
    lax.fori_loop(0, ZCH // NS, zcopy, 0)

    @pl.when(sid < ZCH - (ZCH // NS) * NS)
    def _():
        off = pl.multiple_of(((ZCH // NS) * NS + sid) * B, 8)
        pltpu.sync_copy(r0, acc.at[pl.ds(off, B)])

    @pl.when(sid == 0)
    def _():
        pltpu.sync_copy(r0.at[pl.ds(0, N - ZCH * B)],
                        acc.at[pl.ds(ZCH * B, N - ZCH * B)])

    plsc.subcore_barrier()

    def start_col(g, b):
        pltpu.async_copy(col_hbm.at[_csl(g)], colc[b], isem[b])

    def wait_col(g, b):
        pltpu.make_async_copy(col_hbm.at[_csl(g)], colc[b], isem[b]).wait()

    def start_rowval(g, b):
        pltpu.async_copy(row_hbm.at[_csl(g)], rowc[b], vsem[b])
        pltpu.async_copy(val_hbm.at[_csl(g)], valc[b], vsem[b])

    def wait_rowval(g, b):
        pltpu.make_async_copy(row_hbm.at[_csl(g)], rowc[b], vsem[b]).wait()
        pltpu.make_async_copy(val_hbm.at[_csl(g)], valc[b], vsem[b]).wait()

    def start_gather(g, b):
        pltpu.async_copy(e_hbm.at[colc[b]], rows[b], gsem[b])

    def wait_gather(g, b):
        pltpu.make_async_copy(e_hbm.at[colc[b]], rows[b], gsem[b]).wait()

    def start_scatter(b):
        pltpu.async_copy(rows[b], acc.at[rowc[b]], ssem[b], add=True)

    def wait_scatter(b):
        pltpu.make_async_copy(rows[b], acc.at[rowc[b]], ssem[b]).wait()

    def scale(b, nedge):
        def scale_grp(j, c2, _rb=rows[b], _vc=valc[b]):
            vv = _vc[pl.ds(pl.multiple_of(j * 16, 16), 16)]
            for k in range(16):
                i = j * 16 + k
                v = vv[k]
                for c in range(D // 16):
                    sl = pl.ds(c * 16, 16)
                    _rb[i, sl] = _rb[i, sl] * v
            return c2

        lax.fori_loop(0, nedge // 16, scale_grp, 0)

    for g in range(3):
        start_col(g, g)
    for g in range(2):
        start_rowval(g, g)
    for g in range(2):
        wait_col(g, g)
        start_gather(g, g)

    def step(g, b):
        wait_gather(g, b)
        wait_rowval(g, b)
        scale(b, B)
        start_scatter(b)
        b2 = (b + 2) % DEPTH

        @pl.when(g >= 1)
        def _():
            wait_scatter((b + 2) % DEPTH)

        @pl.when(g + 2 < NCH)
        def _():
            start_rowval(g + 2, b2)
            wait_col(g + 2, b2)
            start_gather(g + 2, b2)

        @pl.when(g + 3 < NCH)
        def _():
            start_col(g + 3, b)

    def pass_body(t, carry):
        for b in range(DEPTH):
            step(t * DEPTH + b, b)
        return carry

    lax.fori_loop(0, NCH // DEPTH, pass_body, 0)
    wait_scatter((NCH - 1) % DEPTH)

    tsl = _csl(NCH, TAIL)
    pltpu.sync_copy(col_hbm.at[tsl], c0.at[pl.ds(0, TAIL)])
    pltpu.sync_copy(row_hbm.at[tsl], trow)
    pltpu.sync_copy(val_hbm.at[tsl], v0.at[pl.ds(0, TAIL)])
    pltpu.async_copy(e_hbm.at[c0.at[pl.ds(0, TAIL)]],
                     r0.at[pl.ds(0, TAIL)], g0).wait()
    vv = v0[pl.ds(0, TAIL)]
    for k in range(TAIL):
        v = vv[k]
        for c in range(D // 16):
            sl = pl.ds(c * 16, 16)
            r0[k, sl] = r0[k, sl] * v
    pltpu.sync_copy(r0.at[pl.ds(0, TAIL)], acc.at[trow], add=True)

    plsc.subcore_barrier()

    off = pl.multiple_of(sid * WB, 16)
    pltpu.sync_copy(acc.at[pl.ds(off, WB)], p_hbm.at[cid, pl.ds(off, WB)])

    @pl.when(sid == NS - 1)
    def _():
        tail = NS * WB
        pltpu.sync_copy(acc.at[pl.ds(tail, N - tail)],
                        p_hbm.at[cid, pl.ds(tail, N - tail)])


@functools.cache
def _get_spmm():
    mesh = plsc.VectorSubcoreMesh(
        core_axis_name="c", subcore_axis_name="s",
        num_cores=NC, num_subcores=NS)
    return pl.kernel(
        _spmm_body,
        out_type=jax.ShapeDtypeStruct((NC, N, D), jnp.float32),
        mesh=mesh,
        scratch_types=(
            [pltpu.VMEM((B, D), jnp.float32) for _ in range(DEPTH)]
            + [pltpu.VMEM((B,), jnp.int32) for _ in range(DEPTH)]
            + [pltpu.VMEM((B,), jnp.int32) for _ in range(DEPTH)]
            + [pltpu.VMEM((B,), jnp.float32) for _ in range(DEPTH)]
            + [pltpu.VMEM((TAIL,), jnp.int32)]
            + [pltpu.VMEM_SHARED((N, D), jnp.float32)]
            + [pltpu.SemaphoreType.DMA for _ in range(4 * DEPTH)]
        ),
    )


_BLK = 400


def _combine_body(p_ref, o_ref):
    o_ref[...] = p_ref[0] + p_ref[1]


_combine = pl.pallas_call(
    _combine_body,
    out_shape=jax.ShapeDtypeStruct((N, D), jnp.float32),
    grid=(N // _BLK,),
    in_specs=[pl.BlockSpec((NC, _BLK, D), lambda i: (0, i, 0))],
    out_specs=pl.BlockSpec((_BLK, D), lambda i: (i, 0)),
)


def _final_body(e0_ref, e1_ref, e2_ref, p_ref, o_ref):
    o_ref[...] = (e0_ref[...] + e1_ref[...] + e2_ref[...]
                  + p_ref[0] + p_ref[1]) * (1.0 / (LAYERS + 1))


_final = pl.pallas_call(
    _final_body,
    out_shape=jax.ShapeDtypeStruct((N, D), jnp.float32),
    grid=(N // _BLK,),
    in_specs=[
        pl.BlockSpec((_BLK, D), lambda i: (i, 0)),
        pl.BlockSpec((_BLK, D), lambda i: (i, 0)),
        pl.BlockSpec((_BLK, D), lambda i: (i, 0)),
        pl.BlockSpec((NC, _BLK, D), lambda i: (0, i, 0)),
    ],
    out_specs=pl.BlockSpec((_BLK, D), lambda i: (i, 0)),
)


def kernel(embedUser, embedItem, graph_row, graph_col, graph_vals):
    rnd = jax.random.uniform(jax.random.key(123), graph_vals.shape)
    keep = (rnd + KEEP_PROB).astype(jnp.int32).astype(bool)
    vals = jnp.where(keep, graph_vals / KEEP_PROB, 0.0)

    e0 = jnp.concatenate([embedUser, embedItem], axis=0)
    col = graph_col.astype(jnp.int32)
    row = graph_row.astype(jnp.int32)

    spmm = _get_spmm()
    e = e0
    embeds = [e0]
    p = None
    for l in range(LAYERS):
        p = spmm(e, col, row, vals)
        if l < LAYERS - 1:
            e = _combine(p)
            embeds.append(e)
    out = _final(embeds[0], embeds[1], embeds[2], p)
    return out[:USERS], out[USERS:]
